# baseline (speedup 1.0000x reference)
import functools

import jax
import jax.numpy as jnp
from jax import lax
from jax.experimental import pallas as pl
from jax.experimental.pallas import tpu as pltpu
from jax.experimental.pallas import tpu_sc as plsc

N = 10000
E = 320000
F_IN = 128
F_OUT = 32
P = 12
FW = 3 * F_OUT

NC, NS = 2, 16
W = NC * NS
CH = 128
GCH = 512
GPW = 21
CPW = 84
EPW = CPW * CH
EP = W * EPW
NPAD = 10240
RPT = NPAD // NS

_mesh = plsc.VectorSubcoreMesh(core_axis_name="c", subcore_axis_name="s")


@functools.partial(
    pl.kernel,
    mesh=_mesh,
    compiler_params=pltpu.CompilerParams(use_tc_tiling_on_sc=False),
    out_type=jax.ShapeDtypeStruct((NC, NPAD, 16), jnp.float32),
    scratch_types=[
        pltpu.VMEM((CPW, CH), jnp.int32),
        pltpu.VMEM((CH, 16), jnp.float32),
        pltpu.VMEM((RPT, 16), jnp.float32),
        pltpu.VMEM_SHARED((NPAD, 16), jnp.float32),
    ],
)
def _deg_kernel(dst_hbm, out_hbm, dstb_v, ones_v, zero_v, acc_sh):
    cid = lax.axis_index("c")
    sid = lax.axis_index("s")
    wid = cid * NS + sid
    pltpu.sync_copy(dst_hbm.at[wid], dstb_v)

    def fill_ones(i, c):
        ones_v[i, :] = jnp.ones((16,), jnp.float32)
        return c
    lax.fori_loop(0, CH, fill_ones, 0)

    def fill_zero(i, c):
        zero_v[i, :] = jnp.zeros((16,), jnp.float32)
        return c
    lax.fori_loop(0, RPT, fill_zero, 0)

    pltpu.sync_copy(zero_v, acc_sh.at[pl.ds(sid * RPT, RPT)])
    plsc.subcore_barrier()

    def chunk(ch, c):
        pltpu.sync_copy(ones_v, acc_sh.at[dstb_v.at[ch]], add=True)
        return c
    lax.fori_loop(0, CPW, chunk, 0)
    plsc.subcore_barrier()

    pltpu.sync_copy(acc_sh.at[pl.ds(sid * RPT, RPT)],
                    out_hbm.at[cid, pl.ds(sid * RPT, RPT)])


HW = FW // 2


@functools.partial(
    pl.kernel,
    mesh=_mesh,
    compiler_params=pltpu.CompilerParams(use_tc_tiling_on_sc=False),
    out_type=jax.ShapeDtypeStruct((NC, 2 * P, NPAD, HW), jnp.float32),
    scratch_types=[
        pltpu.VMEM((GPW, GCH), jnp.int32),
        pltpu.VMEM((CPW, CH), jnp.int32),
        pltpu.VMEM((GCH,), jnp.int32),
        pltpu.VMEM((GCH, HW), jnp.float32),
        pltpu.VMEM((RPT, HW), jnp.float32),
        pltpu.VMEM_SHARED((NPAD, HW), jnp.float32),
        pltpu.SemaphoreType.DMA,
    ],
)
def _agg_kernel(t_hbm, src_hbm, dst_hbm, zeros_hbm, out_hbm,
                srcb_v, dstb_v, idx_v, rows_v, zero_v, acc_sh, gsem):
    cid = lax.axis_index("c")
    sid = lax.axis_index("s")
    wid = cid * NS + sid
    pltpu.sync_copy(src_hbm.at[wid], srcb_v)
    pltpu.sync_copy(dst_hbm.at[wid], dstb_v)
    pltpu.sync_copy(zeros_hbm, zero_v)

    def pass_body(pp, c):
        pltpu.sync_copy(zero_v, acc_sh.at[pl.ds(sid * RPT, RPT)])
        plsc.subcore_barrier()

        def gchunk(g, c2):
            def mk(j, c3):
                idx_v[pl.ds(j * 16, 16)] = srcb_v[g, pl.ds(j * 16, 16)] + pp
                return c3
            lax.fori_loop(0, GCH // 16, mk, 0)
            pltpu.async_copy(t_hbm.at[idx_v], rows_v, gsem).wait()
            for q in range(GCH // CH):
                pltpu.sync_copy(rows_v.at[pl.ds(q * CH, CH)],
                                acc_sh.at[dstb_v.at[g * (GCH // CH) + q]],
                                add=True)
            return c2
        lax.fori_loop(0, GPW, gchunk, 0)
        plsc.subcore_barrier()

        pltpu.sync_copy(acc_sh.at[pl.ds(sid * RPT, RPT)],
                        out_hbm.at[cid, pp, pl.ds(sid * RPT, RPT)])
        plsc.subcore_barrier()
        return c
    lax.fori_loop(0, 2 * P, pass_body, 0)


def _tc_project(xt2, w_all, dinv12):
    RB = 2400

    def body(x_ref, w_ref, d_ref, o_ref):
        t = jnp.dot(x_ref[...], w_ref[...], preferred_element_type=jnp.float32)
        o_ref[...] = t * d_ref[...]

    return pl.pallas_call(
        body,
        grid=(xt2.shape[0] // RB,),
        in_specs=[
            pl.BlockSpec((RB, F_IN), lambda i: (i, 0)),
            pl.BlockSpec((F_IN, FW), lambda i: (0, 0)),
            pl.BlockSpec((RB, 1), lambda i: (i, 0)),
        ],
        out_specs=pl.BlockSpec((RB, FW), lambda i: (i, 0)),
        out_shape=jax.ShapeDtypeStruct((xt2.shape[0], FW), jnp.float32),
    )(xt2, w_all, dinv12)


def _tc_gru(partials, dinv2, att2, ball, U_z, c_z, U_r, c_r, U_h, c_h,
            W_lin, b_lin):
    NB = 1000

    def body(pp_ref, d_ref, att_ref, ball_ref, uz_ref, cz_ref, ur_ref,
             cr_ref, uh_ref, ch_ref, wl_ref, bl_ref, o_ref):
        probs = jax.nn.softmax(att_ref[...], axis=-1)
        dinv = d_ref[...]
        H = jnp.zeros((NB, F_OUT), jnp.float32)
        Hacc = jnp.zeros((NB, F_OUT), jnp.float32)
        for p in range(P):
            raw = jnp.concatenate(
                [pp_ref[0, p, 0] + pp_ref[1, p, 0],
                 pp_ref[0, p, 1] + pp_ref[1, p, 1]], axis=1)
            C = raw * dinv + ball_ref[...]
            Cz = C[:, 0:F_OUT]
            Cr = C[:, F_OUT:2 * F_OUT]
            Chh = C[:, 2 * F_OUT:3 * F_OUT]
            Z = jax.nn.sigmoid(
                jnp.dot(jnp.concatenate([Cz, H], axis=1), uz_ref[...],
                        preferred_element_type=jnp.float32) + cz_ref[...])
            R = jax.nn.sigmoid(
                jnp.dot(jnp.concatenate([Cr, H], axis=1), ur_ref[...],
                        preferred_element_type=jnp.float32) + cr_ref[...])
            Ht = jnp.tanh(
                jnp.dot(jnp.concatenate([Chh, H * R], axis=1), uh_ref[...],
                        preferred_element_type=jnp.float32) + ch_ref[...])
            H = Z * H + (1.0 - Z) * Ht
            Hacc = Hacc + probs[0:1, p:p + 1] * H
        o_ref[...] = (jnp.dot(jnp.maximum(Hacc, 0.0), wl_ref[...],
                              preferred_element_type=jnp.float32)
                      + bl_ref[...])

    return pl.pallas_call(
        body,
        grid=(N // NB,),
        in_specs=[
            pl.BlockSpec((NC, P, 2, NB, HW), lambda i: (0, 0, 0, i, 0)),
            pl.BlockSpec((NB, 1), lambda i: (i, 0)),
            pl.BlockSpec((1, P), lambda i: (0, 0)),
            pl.BlockSpec((1, FW), lambda i: (0, 0)),
            pl.BlockSpec((2 * F_OUT, F_OUT), lambda i: (0, 0)),
            pl.BlockSpec((1, F_OUT), lambda i: (0, 0)),
            pl.BlockSpec((2 * F_OUT, F_OUT), lambda i: (0, 0)),
            pl.BlockSpec((1, F_OUT), lambda i: (0, 0)),
            pl.BlockSpec((2 * F_OUT, F_OUT), lambda i: (0, 0)),
            pl.BlockSpec((1, F_OUT), lambda i: (0, 0)),
            pl.BlockSpec((F_OUT, P), lambda i: (0, 0)),
            pl.BlockSpec((1, P), lambda i: (0, 0)),
        ],
        out_specs=pl.BlockSpec((NB, P), lambda i: (i, 0)),
        out_shape=jax.ShapeDtypeStruct((N, P), jnp.float32),
    )(partials, dinv2, att2, ball, U_z, c_z, U_r, c_r, U_h, c_h,
      W_lin, b_lin)


def kernel(x, edge_index, attention, W_z, b_z, W_r, b_r, W_h, b_h,
           U_z, c_z, U_r, c_r, U_h, c_h, W_lin, b_lin):
    src = edge_index[0].astype(jnp.int32)
    dst = edge_index[1].astype(jnp.int32)
    loop = jnp.arange(N, dtype=jnp.int32)
    pad = EP - E - N
    src_f = jnp.concatenate([src, loop, jnp.zeros((pad,), jnp.int32)])
    dst_f = jnp.concatenate([dst, loop, jnp.full((pad,), N, jnp.int32)])
    src24 = (src_f * (2 * P)).reshape(W, GPW, GCH)
    dstb = dst_f.reshape(W, CPW, CH)

    degp = _deg_kernel(dstb)
    deg = degp[0, :N, 0] + degp[1, :N, 0]
    dinv = lax.rsqrt(deg)
    dinv12 = jnp.repeat(dinv, P)[:, None]

    w_all = jnp.concatenate([W_z, W_r, W_h], axis=1)
    xt2 = jnp.swapaxes(x, 1, 2).reshape(N * P, F_IN)
    t_tab = _tc_project(xt2, w_all, dinv12)
    t_half = t_tab.reshape(N * P * 2, HW)

    zeros_rpt = jnp.zeros((RPT, HW), jnp.float32)
    partials = _agg_kernel(t_half, src24, dstb, zeros_rpt)
    partials = partials.reshape(NC, P, 2, NPAD, HW)

    ball = jnp.concatenate([b_z, b_r, b_h])[None, :]
    out = _tc_gru(partials, dinv[:, None], attention[None, :], ball,
                  U_z, c_z[None, :], U_r, c_r[None, :], U_h, c_h[None, :],
                  W_lin, b_lin[None, :])
    return out


# speedup vs baseline: 1.0001x; 1.0001x over previous
import functools

import jax
import jax.numpy as jnp
from jax import lax
from jax.experimental import pallas as pl
from jax.experimental.pallas import tpu as pltpu
from jax.experimental.pallas import tpu_sc as plsc

N = 10000
E = 320000
F_IN = 128
F_OUT = 32
P = 12
FW = 3 * F_OUT

NC, NS = 2, 16
W = NC * NS
CH = 128
GCH = 512
GPW = 21
CPW = 84
EPW = CPW * CH
EP = W * EPW
NPAD = 10240
RPT = NPAD // NS

_mesh = plsc.VectorSubcoreMesh(core_axis_name="c", subcore_axis_name="s")


@functools.partial(
    pl.kernel,
    mesh=_mesh,
    compiler_params=pltpu.CompilerParams(use_tc_tiling_on_sc=False),
    out_type=jax.ShapeDtypeStruct((NC, NPAD, 16), jnp.float32),
    scratch_types=[
        pltpu.VMEM((CPW, CH), jnp.int32),
        pltpu.VMEM((CH, 16), jnp.float32),
        pltpu.VMEM((RPT, 16), jnp.float32),
        pltpu.VMEM_SHARED((NPAD, 16), jnp.float32),
    ],
)
def _deg_kernel(dst_hbm, out_hbm, dstb_v, ones_v, zero_v, acc_sh):
    cid = lax.axis_index("c")
    sid = lax.axis_index("s")
    wid = cid * NS + sid
    pltpu.sync_copy(dst_hbm.at[wid], dstb_v)

    def fill_ones(i, c):
        ones_v[i, :] = jnp.ones((16,), jnp.float32)
        return c
    lax.fori_loop(0, CH, fill_ones, 0)

    def fill_zero(i, c):
        zero_v[i, :] = jnp.zeros((16,), jnp.float32)
        return c
    lax.fori_loop(0, RPT, fill_zero, 0)

    pltpu.sync_copy(zero_v, acc_sh.at[pl.ds(sid * RPT, RPT)])
    plsc.subcore_barrier()

    def chunk(ch, c):
        pltpu.sync_copy(ones_v, acc_sh.at[dstb_v.at[ch]], add=True)
        return c
    lax.fori_loop(0, CPW, chunk, 0)
    plsc.subcore_barrier()

    pltpu.sync_copy(acc_sh.at[pl.ds(sid * RPT, RPT)],
                    out_hbm.at[cid, pl.ds(sid * RPT, RPT)])


HW = FW // 2


@functools.partial(
    pl.kernel,
    mesh=_mesh,
    compiler_params=pltpu.CompilerParams(use_tc_tiling_on_sc=False),
    out_type=jax.ShapeDtypeStruct((NC, 2 * P, NPAD, HW), jnp.float32),
    scratch_types=[
        pltpu.VMEM((GPW, GCH), jnp.int32),
        pltpu.VMEM((CPW, CH), jnp.int32),
        pltpu.VMEM((GCH,), jnp.int32),
        pltpu.VMEM((GCH, HW), jnp.float32),
        pltpu.VMEM((RPT, HW), jnp.float32),
        pltpu.VMEM_SHARED((NPAD, HW), jnp.float32),
        pltpu.SemaphoreType.DMA,
    ],
)
def _agg_kernel(t_hbm, src_hbm, dst_hbm, zeros_hbm, out_hbm,
                srcb_v, dstb_v, idx_v, rows_v, zero_v, acc_sh, gsem):
    cid = lax.axis_index("c")
    sid = lax.axis_index("s")
    wid = cid * NS + sid
    pltpu.sync_copy(src_hbm.at[wid], srcb_v)
    pltpu.sync_copy(dst_hbm.at[wid], dstb_v)
    pltpu.sync_copy(zeros_hbm, zero_v)

    for pp in range(2 * P):
        pltpu.sync_copy(zero_v, acc_sh.at[pl.ds(sid * RPT, RPT)])
        plsc.subcore_barrier()

        def gchunk(g, c2, pp=pp):
            def mk(j, c3):
                idx_v[pl.ds(j * 16, 16)] = srcb_v[g, pl.ds(j * 16, 16)] + pp
                return c3
            lax.fori_loop(0, GCH // 16, mk, 0)
            pltpu.async_copy(t_hbm.at[idx_v], rows_v, gsem).wait()
            for q in range(GCH // CH):
                pltpu.sync_copy(rows_v.at[pl.ds(q * CH, CH)],
                                acc_sh.at[dstb_v.at[g * (GCH // CH) + q]],
                                add=True)
            return c2
        lax.fori_loop(0, GPW, gchunk, 0)
        plsc.subcore_barrier()

        pltpu.sync_copy(acc_sh.at[pl.ds(sid * RPT, RPT)],
                        out_hbm.at[cid, pp, pl.ds(sid * RPT, RPT)])
        plsc.subcore_barrier()


def _tc_project(xt2, w_all, dinv12):
    RB = 2400

    def body(x_ref, w_ref, d_ref, o_ref):
        t = jnp.dot(x_ref[...], w_ref[...], preferred_element_type=jnp.float32)
        o_ref[...] = t * d_ref[...]

    return pl.pallas_call(
        body,
        grid=(xt2.shape[0] // RB,),
        in_specs=[
            pl.BlockSpec((RB, F_IN), lambda i: (i, 0)),
            pl.BlockSpec((F_IN, FW), lambda i: (0, 0)),
            pl.BlockSpec((RB, 1), lambda i: (i, 0)),
        ],
        out_specs=pl.BlockSpec((RB, FW), lambda i: (i, 0)),
        out_shape=jax.ShapeDtypeStruct((xt2.shape[0], FW), jnp.float32),
    )(xt2, w_all, dinv12)


def _tc_gru(partials, dinv2, att2, ball, U_z, c_z, U_r, c_r, U_h, c_h,
            W_lin, b_lin):
    NB = 1000

    def body(pp_ref, d_ref, att_ref, ball_ref, uz_ref, cz_ref, ur_ref,
             cr_ref, uh_ref, ch_ref, wl_ref, bl_ref, o_ref):
        probs = jax.nn.softmax(att_ref[...], axis=-1)
        dinv = d_ref[...]
        H = jnp.zeros((NB, F_OUT), jnp.float32)
        Hacc = jnp.zeros((NB, F_OUT), jnp.float32)
        for p in range(P):
            raw = jnp.concatenate(
                [pp_ref[0, p, 0] + pp_ref[1, p, 0],
                 pp_ref[0, p, 1] + pp_ref[1, p, 1]], axis=1)
            C = raw * dinv + ball_ref[...]
            Cz = C[:, 0:F_OUT]
            Cr = C[:, F_OUT:2 * F_OUT]
            Chh = C[:, 2 * F_OUT:3 * F_OUT]
            Z = jax.nn.sigmoid(
                jnp.dot(jnp.concatenate([Cz, H], axis=1), uz_ref[...],
                        preferred_element_type=jnp.float32) + cz_ref[...])
            R = jax.nn.sigmoid(
                jnp.dot(jnp.concatenate([Cr, H], axis=1), ur_ref[...],
                        preferred_element_type=jnp.float32) + cr_ref[...])
            Ht = jnp.tanh(
                jnp.dot(jnp.concatenate([Chh, H * R], axis=1), uh_ref[...],
                        preferred_element_type=jnp.float32) + ch_ref[...])
            H = Z * H + (1.0 - Z) * Ht
            Hacc = Hacc + probs[0:1, p:p + 1] * H
        o_ref[...] = (jnp.dot(jnp.maximum(Hacc, 0.0), wl_ref[...],
                              preferred_element_type=jnp.float32)
                      + bl_ref[...])

    return pl.pallas_call(
        body,
        grid=(N // NB,),
        in_specs=[
            pl.BlockSpec((NC, P, 2, NB, HW), lambda i: (0, 0, 0, i, 0)),
            pl.BlockSpec((NB, 1), lambda i: (i, 0)),
            pl.BlockSpec((1, P), lambda i: (0, 0)),
            pl.BlockSpec((1, FW), lambda i: (0, 0)),
            pl.BlockSpec((2 * F_OUT, F_OUT), lambda i: (0, 0)),
            pl.BlockSpec((1, F_OUT), lambda i: (0, 0)),
            pl.BlockSpec((2 * F_OUT, F_OUT), lambda i: (0, 0)),
            pl.BlockSpec((1, F_OUT), lambda i: (0, 0)),
            pl.BlockSpec((2 * F_OUT, F_OUT), lambda i: (0, 0)),
            pl.BlockSpec((1, F_OUT), lambda i: (0, 0)),
            pl.BlockSpec((F_OUT, P), lambda i: (0, 0)),
            pl.BlockSpec((1, P), lambda i: (0, 0)),
        ],
        out_specs=pl.BlockSpec((NB, P), lambda i: (i, 0)),
        out_shape=jax.ShapeDtypeStruct((N, P), jnp.float32),
    )(partials, dinv2, att2, ball, U_z, c_z, U_r, c_r, U_h, c_h,
      W_lin, b_lin)


def kernel(x, edge_index, attention, W_z, b_z, W_r, b_r, W_h, b_h,
           U_z, c_z, U_r, c_r, U_h, c_h, W_lin, b_lin):
    src = edge_index[0].astype(jnp.int32)
    dst = edge_index[1].astype(jnp.int32)
    loop = jnp.arange(N, dtype=jnp.int32)
    pad = EP - E - N
    src_f = jnp.concatenate([src, loop, jnp.zeros((pad,), jnp.int32)])
    dst_f = jnp.concatenate([dst, loop, jnp.full((pad,), N, jnp.int32)])
    src24 = (src_f * (2 * P)).reshape(W, GPW, GCH)
    dstb = dst_f.reshape(W, CPW, CH)

    degp = _deg_kernel(dstb)
    deg = degp[0, :N, 0] + degp[1, :N, 0]
    dinv = lax.rsqrt(deg)
    dinv12 = jnp.repeat(dinv, P)[:, None]

    w_all = jnp.concatenate([W_z, W_r, W_h], axis=1)
    xt2 = jnp.swapaxes(x, 1, 2).reshape(N * P, F_IN)
    t_tab = _tc_project(xt2, w_all, dinv12)
    t_half = t_tab.reshape(N * P * 2, HW)

    zeros_rpt = jnp.zeros((RPT, HW), jnp.float32)
    partials = _agg_kernel(t_half, src24, dstb, zeros_rpt)
    partials = partials.reshape(NC, P, 2, NPAD, HW)

    ball = jnp.concatenate([b_z, b_r, b_h])[None, :]
    out = _tc_gru(partials, dinv[:, None], attention[None, :], ball,
                  U_z, c_z[None, :], U_r, c_r[None, :], U_h, c_h[None, :],
                  W_lin, b_lin[None, :])
    return out


# speedup vs baseline: 1.0012x; 1.0011x over previous
import functools

import jax
import jax.numpy as jnp
from jax import lax
from jax.experimental import pallas as pl
from jax.experimental.pallas import tpu as pltpu
from jax.experimental.pallas import tpu_sc as plsc

N = 10000
E = 320000
F_IN = 128
F_OUT = 32
P = 12
FW = 3 * F_OUT

NC, NS = 2, 16
W = NC * NS
CH = 128
GCH = 512
GPW = 21
CPW = 84
EPW = CPW * CH
EP = W * EPW
NPAD = 10240
RPT = NPAD // NS

_mesh = plsc.VectorSubcoreMesh(core_axis_name="c", subcore_axis_name="s")


@functools.partial(
    pl.kernel,
    mesh=_mesh,
    compiler_params=pltpu.CompilerParams(use_tc_tiling_on_sc=False),
    out_type=jax.ShapeDtypeStruct((NC, NPAD, 16), jnp.float32),
    scratch_types=[
        pltpu.VMEM((CPW, CH), jnp.int32),
        pltpu.VMEM((CH, 16), jnp.float32),
        pltpu.VMEM((RPT, 16), jnp.float32),
        pltpu.VMEM_SHARED((NPAD, 16), jnp.float32),
    ],
)
def _deg_kernel(dst_hbm, out_hbm, dstb_v, ones_v, zero_v, acc_sh):
    cid = lax.axis_index("c")
    sid = lax.axis_index("s")
    wid = cid * NS + sid
    pltpu.sync_copy(dst_hbm.at[wid], dstb_v)

    def fill_ones(i, c):
        ones_v[i, :] = jnp.ones((16,), jnp.float32)
        return c
    lax.fori_loop(0, CH, fill_ones, 0)

    def fill_zero(i, c):
        zero_v[i, :] = jnp.zeros((16,), jnp.float32)
        return c
    lax.fori_loop(0, RPT, fill_zero, 0)

    pltpu.sync_copy(zero_v, acc_sh.at[pl.ds(sid * RPT, RPT)])
    plsc.subcore_barrier()

    def chunk(ch, c):
        pltpu.sync_copy(ones_v, acc_sh.at[dstb_v.at[ch]], add=True)
        return c
    lax.fori_loop(0, CPW, chunk, 0)
    plsc.subcore_barrier()

    pltpu.sync_copy(acc_sh.at[pl.ds(sid * RPT, RPT)],
                    out_hbm.at[cid, pl.ds(sid * RPT, RPT)])


HW = FW // 2


@functools.partial(
    pl.kernel,
    mesh=_mesh,
    compiler_params=pltpu.CompilerParams(use_tc_tiling_on_sc=False),
    out_type=jax.ShapeDtypeStruct((NC, 2 * P, NPAD, HW), jnp.float32),
    scratch_types=[
        pltpu.VMEM((GPW, GCH), jnp.int32),
        pltpu.VMEM((CPW, CH), jnp.int32),
        pltpu.VMEM((GCH,), jnp.int32),
        pltpu.VMEM((GCH, HW), jnp.float32),
        pltpu.VMEM((RPT, HW), jnp.float32),
        pltpu.VMEM_SHARED((NPAD, HW), jnp.float32),
        pltpu.SemaphoreType.DMA,
    ],
)
def _agg_kernel(t_hbm, src_hbm, dst_hbm, zeros_hbm, out_hbm,
                srcb_v, dstb_v, idx_v, rows_v, zero_v, acc_sh, gsem):
    cid = lax.axis_index("c")
    sid = lax.axis_index("s")
    wid = cid * NS + sid
    pltpu.sync_copy(src_hbm.at[wid], srcb_v)
    pltpu.sync_copy(dst_hbm.at[wid], dstb_v)
    pltpu.sync_copy(zeros_hbm, zero_v)

    for pp in range(2 * P):
        pltpu.sync_copy(zero_v, acc_sh.at[pl.ds(sid * RPT, RPT)])
        plsc.subcore_barrier()

        def gchunk(g, c2, pp=pp):
            def mk(j, c3):
                idx_v[pl.ds(j * 16, 16)] = srcb_v[g, pl.ds(j * 16, 16)] + pp
                return c3
            lax.fori_loop(0, GCH // 16, mk, 0)
            pltpu.async_copy(t_hbm.at[idx_v], rows_v, gsem).wait()
            for q in range(GCH // CH):
                pltpu.sync_copy(rows_v.at[pl.ds(q * CH, CH)],
                                acc_sh.at[dstb_v.at[g * (GCH // CH) + q]],
                                add=True)
            return c2
        lax.fori_loop(0, GPW, gchunk, 0)
        plsc.subcore_barrier()

        pltpu.sync_copy(acc_sh.at[pl.ds(sid * RPT, RPT)],
                        out_hbm.at[cid, pp, pl.ds(sid * RPT, RPT)])
        plsc.subcore_barrier()


def _tc_project(xt2, w_all, dinv12):
    RB = 2400

    def body(x_ref, w_ref, d_ref, o_ref):
        t = jnp.dot(x_ref[...], w_ref[...], preferred_element_type=jnp.float32)
        o_ref[...] = t * d_ref[...]

    return pl.pallas_call(
        body,
        grid=(xt2.shape[0] // RB,),
        in_specs=[
            pl.BlockSpec((RB, F_IN), lambda i: (i, 0)),
            pl.BlockSpec((F_IN, FW), lambda i: (0, 0)),
            pl.BlockSpec((RB, 1), lambda i: (i, 0)),
        ],
        out_specs=pl.BlockSpec((RB, FW), lambda i: (i, 0)),
        out_shape=jax.ShapeDtypeStruct((xt2.shape[0], FW), jnp.float32),
    )(xt2, w_all, dinv12)


def _tc_gru(partials, dinv2, att2, ball, U_z, c_z, U_r, c_r, U_h, c_h,
            W_lin, b_lin):
    NB = 1000

    def body(pp_ref, d_ref, att_ref, ball_ref, uz_ref, cz_ref, ur_ref,
             cr_ref, uh_ref, ch_ref, wl_ref, bl_ref, o_ref):
        probs = jax.nn.softmax(att_ref[...], axis=-1)
        dinv = d_ref[...]
        H = jnp.zeros((NB, F_OUT), jnp.float32)
        Hacc = jnp.zeros((NB, F_OUT), jnp.float32)
        for p in range(P):
            raw = jnp.concatenate(
                [pp_ref[0, p, 0] + pp_ref[1, p, 0],
                 pp_ref[0, p, 1] + pp_ref[1, p, 1]], axis=1)
            C = raw * dinv + ball_ref[...]
            Cz = C[:, 0:F_OUT]
            Cr = C[:, F_OUT:2 * F_OUT]
            Chh = C[:, 2 * F_OUT:3 * F_OUT]
            Z = jax.nn.sigmoid(
                jnp.dot(jnp.concatenate([Cz, H], axis=1), uz_ref[...],
                        preferred_element_type=jnp.float32) + cz_ref[...])
            R = jax.nn.sigmoid(
                jnp.dot(jnp.concatenate([Cr, H], axis=1), ur_ref[...],
                        preferred_element_type=jnp.float32) + cr_ref[...])
            Ht = jnp.tanh(
                jnp.dot(jnp.concatenate([Chh, H * R], axis=1), uh_ref[...],
                        preferred_element_type=jnp.float32) + ch_ref[...])
            H = Z * H + (1.0 - Z) * Ht
            Hacc = Hacc + probs[0:1, p:p + 1] * H
        o_ref[...] = (jnp.dot(jnp.maximum(Hacc, 0.0), wl_ref[...],
                              preferred_element_type=jnp.float32)
                      + bl_ref[...])

    return pl.pallas_call(
        body,
        grid=(N // NB,),
        in_specs=[
            pl.BlockSpec((NC, P, 2, NB, HW), lambda i: (0, 0, 0, i, 0)),
            pl.BlockSpec((NB, 1), lambda i: (i, 0)),
            pl.BlockSpec((1, P), lambda i: (0, 0)),
            pl.BlockSpec((1, FW), lambda i: (0, 0)),
            pl.BlockSpec((2 * F_OUT, F_OUT), lambda i: (0, 0)),
            pl.BlockSpec((1, F_OUT), lambda i: (0, 0)),
            pl.BlockSpec((2 * F_OUT, F_OUT), lambda i: (0, 0)),
            pl.BlockSpec((1, F_OUT), lambda i: (0, 0)),
            pl.BlockSpec((2 * F_OUT, F_OUT), lambda i: (0, 0)),
            pl.BlockSpec((1, F_OUT), lambda i: (0, 0)),
            pl.BlockSpec((F_OUT, P), lambda i: (0, 0)),
            pl.BlockSpec((1, P), lambda i: (0, 0)),
        ],
        out_specs=pl.BlockSpec((NB, P), lambda i: (i, 0)),
        out_shape=jax.ShapeDtypeStruct((N, P), jnp.float32),
    )(partials, dinv2, att2, ball, U_z, c_z, U_r, c_r, U_h, c_h,
      W_lin, b_lin)


def kernel(x, edge_index, attention, W_z, b_z, W_r, b_r, W_h, b_h,
           U_z, c_z, U_r, c_r, U_h, c_h, W_lin, b_lin):
    src = edge_index[0].astype(jnp.int32)
    dst = edge_index[1].astype(jnp.int32)
    loop = jnp.arange(N, dtype=jnp.int32)
    pad = EP - E - N
    src_f = jnp.concatenate([src, loop, jnp.zeros((pad,), jnp.int32)])
    dst_pad = N + jnp.arange(pad, dtype=jnp.int32) % (NPAD - N)
    dst_f = jnp.concatenate([dst, loop, dst_pad])
    src24 = (src_f * (2 * P)).reshape(W, GPW, GCH)
    dstb = dst_f.reshape(W, CPW, CH)

    degp = _deg_kernel(dstb)
    deg = degp[0, :N, 0] + degp[1, :N, 0]
    dinv = lax.rsqrt(deg)
    dinv12 = jnp.repeat(dinv, P)[:, None]

    w_all = jnp.concatenate([W_z, W_r, W_h], axis=1)
    xt2 = jnp.swapaxes(x, 1, 2).reshape(N * P, F_IN)
    t_tab = _tc_project(xt2, w_all, dinv12)
    t_half = t_tab.reshape(N * P * 2, HW)

    zeros_rpt = jnp.zeros((RPT, HW), jnp.float32)
    partials = _agg_kernel(t_half, src24, dstb, zeros_rpt)
    partials = partials.reshape(NC, P, 2, NPAD, HW)

    ball = jnp.concatenate([b_z, b_r, b_h])[None, :]
    out = _tc_gru(partials, dinv[:, None], attention[None, :], ball,
                  U_z, c_z[None, :], U_r, c_r[None, :], U_h, c_h[None, :],
                  W_lin, b_lin[None, :])
    return out


# speedup vs baseline: 2.0297x; 2.0273x over previous
import functools

import jax
import jax.numpy as jnp
from jax import lax
from jax.experimental import pallas as pl
from jax.experimental.pallas import tpu as pltpu
from jax.experimental.pallas import tpu_sc as plsc

N = 10000
E = 320000
F_IN = 128
F_OUT = 32
P = 12
FW = 3 * F_OUT

NC, NS = 2, 16
W = NC * NS
CH = 128
CPW = 81
EPW = CPW * CH
EP = W * EPW
NPAD = 10240
RPT = NPAD // NS

_mesh = plsc.VectorSubcoreMesh(core_axis_name="c", subcore_axis_name="s")


@functools.partial(
    pl.kernel,
    mesh=_mesh,
    compiler_params=pltpu.CompilerParams(use_tc_tiling_on_sc=False),
    out_type=jax.ShapeDtypeStruct((NC, NPAD, 16), jnp.float32),
    scratch_types=[
        pltpu.VMEM((CPW, CH), jnp.int32),
        pltpu.VMEM((CH, 16), jnp.float32),
        pltpu.VMEM((RPT, 16), jnp.float32),
        pltpu.VMEM_SHARED((NPAD, 16), jnp.float32),
    ],
)
def _deg_kernel(dst_hbm, out_hbm, dstb_v, ones_v, zero_v, acc_sh):
    cid = lax.axis_index("c")
    sid = lax.axis_index("s")
    wid = cid * NS + sid
    pltpu.sync_copy(dst_hbm.at[wid], dstb_v)

    def fill_ones(i, c):
        ones_v[i, :] = jnp.ones((16,), jnp.float32)
        return c
    lax.fori_loop(0, CH, fill_ones, 0)

    def fill_zero(i, c):
        zero_v[i, :] = jnp.zeros((16,), jnp.float32)
        return c
    lax.fori_loop(0, RPT, fill_zero, 0)

    pltpu.sync_copy(zero_v, acc_sh.at[pl.ds(sid * RPT, RPT)])
    plsc.subcore_barrier()

    def chunk(ch, c):
        pltpu.sync_copy(ones_v, acc_sh.at[dstb_v.at[ch]], add=True)
        return c
    lax.fori_loop(0, CPW, chunk, 0)
    plsc.subcore_barrier()

    pltpu.sync_copy(acc_sh.at[pl.ds(sid * RPT, RPT)],
                    out_hbm.at[cid, pl.ds(sid * RPT, RPT)])


HW = FW // 2


@functools.partial(
    pl.kernel,
    mesh=_mesh,
    compiler_params=pltpu.CompilerParams(use_tc_tiling_on_sc=False),
    out_type=jax.ShapeDtypeStruct((NC, P, 2, NPAD, HW), jnp.float32),
    scratch_types=[
        pltpu.VMEM((CPW, CH), jnp.int32),
        pltpu.VMEM((CPW, CH), jnp.int32),
        pltpu.VMEM((CH,), jnp.int32),
        pltpu.VMEM((CH, HW), jnp.float32),
        pltpu.VMEM((RPT, HW), jnp.float32),
        pltpu.VMEM_SHARED((NPAD, HW), jnp.float32),
        pltpu.SemaphoreType.DMA,
    ],
)
def _agg_kernel(t_hbm, src_hbm, dst_hbm, zeros_hbm, out_hbm,
                srcb_v, dstb_v, idx_v, rows_v, zero_v, acc_sh, sem):
    cid = lax.axis_index("c")
    sid = lax.axis_index("s")
    wid = cid * NS + sid
    pltpu.sync_copy(src_hbm.at[wid], srcb_v)
    pltpu.sync_copy(dst_hbm.at[wid], dstb_v)
    pltpu.sync_copy(zeros_hbm, zero_v)

    for p in range(P):
        for h in range(2):
            off = 2 * p + h
            pltpu.sync_copy(zero_v, acc_sh.at[pl.ds(sid * RPT, RPT)])
            plsc.subcore_barrier()

            def chunk(ch, c, off=off):
                def mk(j, c2):
                    idx_v[pl.ds(j * 16, 16)] = (
                        srcb_v[ch, pl.ds(j * 16, 16)] + off)
                    return c2
                lax.fori_loop(0, CH // 16, mk, 0)
                pltpu.async_copy(t_hbm.at[idx_v], rows_v, sem).wait()
                pltpu.sync_copy(rows_v, acc_sh.at[dstb_v.at[ch]], add=True)
                return c
            lax.fori_loop(0, CPW, chunk, 0)
            plsc.subcore_barrier()

            pltpu.sync_copy(acc_sh.at[pl.ds(sid * RPT, RPT)],
                            out_hbm.at[cid, p, h, pl.ds(sid * RPT, RPT)])
            plsc.subcore_barrier()


def _tc_project(xt2, w_all, dinv12):
    RB = 2400

    def body(x_ref, w_ref, d_ref, o_ref):
        t = jnp.dot(x_ref[...], w_ref[...], preferred_element_type=jnp.float32)
        o_ref[...] = t * d_ref[...]

    return pl.pallas_call(
        body,
        grid=(xt2.shape[0] // RB,),
        in_specs=[
            pl.BlockSpec((RB, F_IN), lambda i: (i, 0)),
            pl.BlockSpec((F_IN, FW), lambda i: (0, 0)),
            pl.BlockSpec((RB, 1), lambda i: (i, 0)),
        ],
        out_specs=pl.BlockSpec((RB, FW), lambda i: (i, 0)),
        out_shape=jax.ShapeDtypeStruct((xt2.shape[0], FW), jnp.float32),
    )(xt2, w_all, dinv12)


def _tc_gru(partials, dinv2, att2, ball, U_z, c_z, U_r, c_r, U_h, c_h,
            W_lin, b_lin):
    NB = 1000

    def body(pp_ref, d_ref, att_ref, ball_ref, uz_ref, cz_ref, ur_ref,
             cr_ref, uh_ref, ch_ref, wl_ref, bl_ref, o_ref):
        probs = jax.nn.softmax(att_ref[...], axis=-1)
        dinv = d_ref[...]
        H = jnp.zeros((NB, F_OUT), jnp.float32)
        Hacc = jnp.zeros((NB, F_OUT), jnp.float32)
        for p in range(P):
            raw = jnp.concatenate(
                [pp_ref[0, p, 0] + pp_ref[1, p, 0],
                 pp_ref[0, p, 1] + pp_ref[1, p, 1]], axis=1)
            C = raw * dinv + ball_ref[...]
            Cz = C[:, 0:F_OUT]
            Cr = C[:, F_OUT:2 * F_OUT]
            Chh = C[:, 2 * F_OUT:3 * F_OUT]
            Z = jax.nn.sigmoid(
                jnp.dot(jnp.concatenate([Cz, H], axis=1), uz_ref[...],
                        preferred_element_type=jnp.float32) + cz_ref[...])
            R = jax.nn.sigmoid(
                jnp.dot(jnp.concatenate([Cr, H], axis=1), ur_ref[...],
                        preferred_element_type=jnp.float32) + cr_ref[...])
            Ht = jnp.tanh(
                jnp.dot(jnp.concatenate([Chh, H * R], axis=1), uh_ref[...],
                        preferred_element_type=jnp.float32) + ch_ref[...])
            H = Z * H + (1.0 - Z) * Ht
            Hacc = Hacc + probs[0:1, p:p + 1] * H
        o_ref[...] = (jnp.dot(jnp.maximum(Hacc, 0.0), wl_ref[...],
                              preferred_element_type=jnp.float32)
                      + bl_ref[...])

    return pl.pallas_call(
        body,
        grid=(N // NB,),
        in_specs=[
            pl.BlockSpec((NC, P, 2, NB, HW), lambda i: (0, 0, 0, i, 0)),
            pl.BlockSpec((NB, 1), lambda i: (i, 0)),
            pl.BlockSpec((1, P), lambda i: (0, 0)),
            pl.BlockSpec((1, FW), lambda i: (0, 0)),
            pl.BlockSpec((2 * F_OUT, F_OUT), lambda i: (0, 0)),
            pl.BlockSpec((1, F_OUT), lambda i: (0, 0)),
            pl.BlockSpec((2 * F_OUT, F_OUT), lambda i: (0, 0)),
            pl.BlockSpec((1, F_OUT), lambda i: (0, 0)),
            pl.BlockSpec((2 * F_OUT, F_OUT), lambda i: (0, 0)),
            pl.BlockSpec((1, F_OUT), lambda i: (0, 0)),
            pl.BlockSpec((F_OUT, P), lambda i: (0, 0)),
            pl.BlockSpec((1, P), lambda i: (0, 0)),
        ],
        out_specs=pl.BlockSpec((NB, P), lambda i: (i, 0)),
        out_shape=jax.ShapeDtypeStruct((N, P), jnp.float32),
    )(partials, dinv2, att2, ball, U_z, c_z, U_r, c_r, U_h, c_h,
      W_lin, b_lin)


def kernel(x, edge_index, attention, W_z, b_z, W_r, b_r, W_h, b_h,
           U_z, c_z, U_r, c_r, U_h, c_h, W_lin, b_lin):
    src = edge_index[0].astype(jnp.int32)
    dst = edge_index[1].astype(jnp.int32)
    loop = jnp.arange(N, dtype=jnp.int32)
    pad = EP - E - N
    src_f = jnp.concatenate([src, loop, jnp.zeros((pad,), jnp.int32)])
    dst_pad = N + jnp.arange(pad, dtype=jnp.int32) % (NPAD - N)
    dst_f = jnp.concatenate([dst, loop, dst_pad])
    src24 = (src_f * (2 * P)).reshape(W, CPW, CH)
    dstb = dst_f.reshape(W, CPW, CH)

    degp = _deg_kernel(dstb)
    deg = degp[0, :N, 0] + degp[1, :N, 0]
    dinv = lax.rsqrt(deg)
    dinv12 = jnp.repeat(dinv, P)[:, None]

    w_all = jnp.concatenate([W_z, W_r, W_h], axis=1)
    xt2 = jnp.swapaxes(x, 1, 2).reshape(N * P, F_IN)
    t_tab = _tc_project(xt2, w_all, dinv12)
    t_half = t_tab.reshape(N * P * 2, HW)

    zeros_rpt = jnp.zeros((RPT, HW), jnp.float32)
    partials = _agg_kernel(t_half, src24, dstb, zeros_rpt)

    ball = jnp.concatenate([b_z, b_r, b_h])[None, :]
    out = _tc_gru(partials, dinv[:, None], attention[None, :], ball,
                  U_z, c_z[None, :], U_r, c_r[None, :], U_h, c_h[None, :],
                  W_lin, b_lin[None, :])
    return out


# speedup vs baseline: 2.3711x; 1.1682x over previous
import functools

import jax
import jax.numpy as jnp
from jax import lax
from jax.experimental import pallas as pl
from jax.experimental.pallas import tpu as pltpu
from jax.experimental.pallas import tpu_sc as plsc

N = 10000
E = 320000
F_IN = 128
F_OUT = 32
P = 12
FW = 3 * F_OUT

NC, NS = 2, 16
W = NC * NS
CH = 128
CPW = 81
EPW = CPW * CH
EP = W * EPW
NPAD = 10240
RPT = NPAD // NS

_mesh = plsc.VectorSubcoreMesh(core_axis_name="c", subcore_axis_name="s")


@functools.partial(
    pl.kernel,
    mesh=_mesh,
    compiler_params=pltpu.CompilerParams(use_tc_tiling_on_sc=False),
    out_type=jax.ShapeDtypeStruct((NC, NPAD, 16), jnp.float32),
    scratch_types=[
        pltpu.VMEM((CPW, CH), jnp.int32),
        pltpu.VMEM((CH, 16), jnp.float32),
        pltpu.VMEM((RPT, 16), jnp.float32),
        pltpu.VMEM_SHARED((NPAD, 16), jnp.float32),
    ],
)
def _deg_kernel(dst_hbm, out_hbm, dstb_v, ones_v, zero_v, acc_sh):
    cid = lax.axis_index("c")
    sid = lax.axis_index("s")
    wid = cid * NS + sid
    pltpu.sync_copy(dst_hbm.at[wid], dstb_v)

    def fill_ones(i, c):
        ones_v[i, :] = jnp.ones((16,), jnp.float32)
        return c
    lax.fori_loop(0, CH, fill_ones, 0)

    def fill_zero(i, c):
        zero_v[i, :] = jnp.zeros((16,), jnp.float32)
        return c
    lax.fori_loop(0, RPT, fill_zero, 0)

    pltpu.sync_copy(zero_v, acc_sh.at[pl.ds(sid * RPT, RPT)])
    plsc.subcore_barrier()

    def chunk(ch, c):
        pltpu.sync_copy(ones_v, acc_sh.at[dstb_v.at[ch]], add=True)
        return c
    lax.fori_loop(0, CPW, chunk, 0)
    plsc.subcore_barrier()

    pltpu.sync_copy(acc_sh.at[pl.ds(sid * RPT, RPT)],
                    out_hbm.at[cid, pl.ds(sid * RPT, RPT)])


HW = FW // 2


@functools.partial(
    pl.kernel,
    mesh=_mesh,
    compiler_params=pltpu.CompilerParams(use_tc_tiling_on_sc=False),
    out_type=jax.ShapeDtypeStruct((NC, P, 2, NPAD, HW), jnp.float32),
    scratch_types=[
        pltpu.VMEM((CPW, CH), jnp.int32),
        pltpu.VMEM((CPW, CH), jnp.int32),
        pltpu.VMEM((CH,), jnp.int32),
        pltpu.VMEM((CH,), jnp.int32),
        pltpu.VMEM((CH, HW), jnp.float32),
        pltpu.VMEM((CH, HW), jnp.float32),
        pltpu.VMEM((RPT, HW), jnp.float32),
        pltpu.VMEM_SHARED((NPAD, HW), jnp.float32),
        pltpu.SemaphoreType.DMA,
    ],
)
def _agg_kernel(t_hbm, src_hbm, dst_hbm, zeros_hbm, out_hbm,
                srcb_v, dstb_v, idx_a, idx_b, rows_a, rows_b, zero_v,
                acc_sh, sem):
    cid = lax.axis_index("c")
    sid = lax.axis_index("s")
    wid = cid * NS + sid
    pltpu.sync_copy(src_hbm.at[wid], srcb_v)
    pltpu.sync_copy(dst_hbm.at[wid], dstb_v)
    pltpu.sync_copy(zeros_hbm, zero_v)

    def gstart(idx, rows, ch, off):
        def mk(j, c):
            idx[pl.ds(j * 16, 16)] = srcb_v[ch, pl.ds(j * 16, 16)] + off
            return c
        lax.fori_loop(0, CH // 16, mk, 0)
        pltpu.async_copy(t_hbm.at[idx], rows, sem)

    def gwait(rows):
        pltpu.make_async_copy(t_hbm.at[pl.ds(0, CH)], rows, sem).wait()

    for p in range(P):
        for h in range(2):
            off = 2 * p + h
            pltpu.sync_copy(zero_v, acc_sh.at[pl.ds(sid * RPT, RPT)])
            plsc.subcore_barrier()

            gstart(idx_a, rows_a, 0, off)

            def pair(i, c, off=off):
                ch = 2 * i
                gwait(rows_a)
                gstart(idx_b, rows_b, ch + 1, off)
                pltpu.sync_copy(rows_a, acc_sh.at[dstb_v.at[ch]], add=True)
                gwait(rows_b)
                gstart(idx_a, rows_a, ch + 2, off)
                pltpu.sync_copy(rows_b, acc_sh.at[dstb_v.at[ch + 1]],
                                add=True)
                return c
            lax.fori_loop(0, (CPW - 1) // 2, pair, 0)

            gwait(rows_a)
            pltpu.sync_copy(rows_a, acc_sh.at[dstb_v.at[CPW - 1]], add=True)
            plsc.subcore_barrier()

            pltpu.sync_copy(acc_sh.at[pl.ds(sid * RPT, RPT)],
                            out_hbm.at[cid, p, h, pl.ds(sid * RPT, RPT)])
            plsc.subcore_barrier()


def _tc_project(xt2, w_all, dinv12):
    RB = 2400

    def body(x_ref, w_ref, d_ref, o_ref):
        t = jnp.dot(x_ref[...], w_ref[...], preferred_element_type=jnp.float32)
        o_ref[...] = t * d_ref[...]

    return pl.pallas_call(
        body,
        grid=(xt2.shape[0] // RB,),
        in_specs=[
            pl.BlockSpec((RB, F_IN), lambda i: (i, 0)),
            pl.BlockSpec((F_IN, FW), lambda i: (0, 0)),
            pl.BlockSpec((RB, 1), lambda i: (i, 0)),
        ],
        out_specs=pl.BlockSpec((RB, FW), lambda i: (i, 0)),
        out_shape=jax.ShapeDtypeStruct((xt2.shape[0], FW), jnp.float32),
    )(xt2, w_all, dinv12)


def _tc_gru(partials, dinv2, att2, ball, U_z, c_z, U_r, c_r, U_h, c_h,
            W_lin, b_lin):
    NB = 1000

    def body(pp_ref, d_ref, att_ref, ball_ref, uz_ref, cz_ref, ur_ref,
             cr_ref, uh_ref, ch_ref, wl_ref, bl_ref, o_ref):
        probs = jax.nn.softmax(att_ref[...], axis=-1)
        dinv = d_ref[...]
        H = jnp.zeros((NB, F_OUT), jnp.float32)
        Hacc = jnp.zeros((NB, F_OUT), jnp.float32)
        for p in range(P):
            raw = jnp.concatenate(
                [pp_ref[0, p, 0] + pp_ref[1, p, 0],
                 pp_ref[0, p, 1] + pp_ref[1, p, 1]], axis=1)
            C = raw * dinv + ball_ref[...]
            Cz = C[:, 0:F_OUT]
            Cr = C[:, F_OUT:2 * F_OUT]
            Chh = C[:, 2 * F_OUT:3 * F_OUT]
            Z = jax.nn.sigmoid(
                jnp.dot(jnp.concatenate([Cz, H], axis=1), uz_ref[...],
                        preferred_element_type=jnp.float32) + cz_ref[...])
            R = jax.nn.sigmoid(
                jnp.dot(jnp.concatenate([Cr, H], axis=1), ur_ref[...],
                        preferred_element_type=jnp.float32) + cr_ref[...])
            Ht = jnp.tanh(
                jnp.dot(jnp.concatenate([Chh, H * R], axis=1), uh_ref[...],
                        preferred_element_type=jnp.float32) + ch_ref[...])
            H = Z * H + (1.0 - Z) * Ht
            Hacc = Hacc + probs[0:1, p:p + 1] * H
        o_ref[...] = (jnp.dot(jnp.maximum(Hacc, 0.0), wl_ref[...],
                              preferred_element_type=jnp.float32)
                      + bl_ref[...])

    return pl.pallas_call(
        body,
        grid=(N // NB,),
        in_specs=[
            pl.BlockSpec((NC, P, 2, NB, HW), lambda i: (0, 0, 0, i, 0)),
            pl.BlockSpec((NB, 1), lambda i: (i, 0)),
            pl.BlockSpec((1, P), lambda i: (0, 0)),
            pl.BlockSpec((1, FW), lambda i: (0, 0)),
            pl.BlockSpec((2 * F_OUT, F_OUT), lambda i: (0, 0)),
            pl.BlockSpec((1, F_OUT), lambda i: (0, 0)),
            pl.BlockSpec((2 * F_OUT, F_OUT), lambda i: (0, 0)),
            pl.BlockSpec((1, F_OUT), lambda i: (0, 0)),
            pl.BlockSpec((2 * F_OUT, F_OUT), lambda i: (0, 0)),
            pl.BlockSpec((1, F_OUT), lambda i: (0, 0)),
            pl.BlockSpec((F_OUT, P), lambda i: (0, 0)),
            pl.BlockSpec((1, P), lambda i: (0, 0)),
        ],
        out_specs=pl.BlockSpec((NB, P), lambda i: (i, 0)),
        out_shape=jax.ShapeDtypeStruct((N, P), jnp.float32),
    )(partials, dinv2, att2, ball, U_z, c_z, U_r, c_r, U_h, c_h,
      W_lin, b_lin)


def kernel(x, edge_index, attention, W_z, b_z, W_r, b_r, W_h, b_h,
           U_z, c_z, U_r, c_r, U_h, c_h, W_lin, b_lin):
    src = edge_index[0].astype(jnp.int32)
    dst = edge_index[1].astype(jnp.int32)
    loop = jnp.arange(N, dtype=jnp.int32)
    pad = EP - E - N
    src_f = jnp.concatenate([src, loop, jnp.zeros((pad,), jnp.int32)])
    dst_pad = N + jnp.arange(pad, dtype=jnp.int32) % (NPAD - N)
    dst_f = jnp.concatenate([dst, loop, dst_pad])
    src24 = (src_f * (2 * P)).reshape(W, CPW, CH)
    dstb = dst_f.reshape(W, CPW, CH)

    degp = _deg_kernel(dstb)
    deg = degp[0, :N, 0] + degp[1, :N, 0]
    dinv = lax.rsqrt(deg)
    dinv12 = jnp.repeat(dinv, P)[:, None]

    w_all = jnp.concatenate([W_z, W_r, W_h], axis=1)
    xt2 = jnp.swapaxes(x, 1, 2).reshape(N * P, F_IN)
    t_tab = _tc_project(xt2, w_all, dinv12)
    t_half = t_tab.reshape(N * P * 2, HW)

    zeros_rpt = jnp.zeros((RPT, HW), jnp.float32)
    partials = _agg_kernel(t_half, src24, dstb, zeros_rpt)

    ball = jnp.concatenate([b_z, b_r, b_h])[None, :]
    out = _tc_gru(partials, dinv[:, None], attention[None, :], ball,
                  U_z, c_z[None, :], U_r, c_r[None, :], U_h, c_h[None, :],
                  W_lin, b_lin[None, :])
    return out


# speedup vs baseline: 2.3913x; 1.0085x over previous
import functools

import jax
import jax.numpy as jnp
from jax import lax
from jax.experimental import pallas as pl
from jax.experimental.pallas import tpu as pltpu
from jax.experimental.pallas import tpu_sc as plsc

N = 10000
E = 320000
F_IN = 128
F_OUT = 32
P = 12
FW = 3 * F_OUT

NC, NS = 2, 16
W = NC * NS
CH = 128
CPW = 81
EPW = CPW * CH
EP = W * EPW
NPAD = 10240
RPT = NPAD // NS

_mesh = plsc.VectorSubcoreMesh(core_axis_name="c", subcore_axis_name="s")


@functools.partial(
    pl.kernel,
    mesh=_mesh,
    compiler_params=pltpu.CompilerParams(use_tc_tiling_on_sc=False),
    out_type=jax.ShapeDtypeStruct((NC, NPAD, 16), jnp.float32),
    scratch_types=[
        pltpu.VMEM((CPW, CH), jnp.int32),
        pltpu.VMEM((CH, 16), jnp.float32),
        pltpu.VMEM((RPT, 16), jnp.float32),
        pltpu.VMEM_SHARED((NPAD, 16), jnp.float32),
    ],
)
def _deg_kernel(dst_hbm, out_hbm, dstb_v, ones_v, zero_v, acc_sh):
    cid = lax.axis_index("c")
    sid = lax.axis_index("s")
    wid = cid * NS + sid
    pltpu.sync_copy(dst_hbm.at[wid], dstb_v)

    def fill_ones(i, c):
        ones_v[i, :] = jnp.ones((16,), jnp.float32)
        return c
    lax.fori_loop(0, CH, fill_ones, 0)

    def fill_zero(i, c):
        zero_v[i, :] = jnp.zeros((16,), jnp.float32)
        return c
    lax.fori_loop(0, RPT, fill_zero, 0)

    pltpu.sync_copy(zero_v, acc_sh.at[pl.ds(sid * RPT, RPT)])
    plsc.subcore_barrier()

    def chunk(ch, c):
        pltpu.sync_copy(ones_v, acc_sh.at[dstb_v.at[ch]], add=True)
        return c
    lax.fori_loop(0, CPW, chunk, 0)
    plsc.subcore_barrier()

    pltpu.sync_copy(acc_sh.at[pl.ds(sid * RPT, RPT)],
                    out_hbm.at[cid, pl.ds(sid * RPT, RPT)])


HW = FW // 2


@functools.partial(
    pl.kernel,
    mesh=_mesh,
    compiler_params=pltpu.CompilerParams(use_tc_tiling_on_sc=False),
    out_type=jax.ShapeDtypeStruct((NC, P, 2, NPAD, HW), jnp.float32),
    scratch_types=[
        pltpu.VMEM((CPW, CH), jnp.int32),
        pltpu.VMEM((CPW, CH), jnp.int32),
        pltpu.VMEM((CH,), jnp.int32),
        pltpu.VMEM((CH,), jnp.int32),
        pltpu.VMEM((CH, HW), jnp.float32),
        pltpu.VMEM((CH, HW), jnp.float32),
        pltpu.VMEM((RPT, HW), jnp.float32),
        pltpu.VMEM_SHARED((NPAD, HW), jnp.float32),
        pltpu.SemaphoreType.DMA,
    ],
)
def _agg_kernel(t_hbm, src_hbm, dst_hbm, zeros_hbm, out_hbm,
                srcb_v, dstb_v, idx_a, idx_b, rows_a, rows_b, zero_v,
                acc_sh, sem):
    cid = lax.axis_index("c")
    sid = lax.axis_index("s")
    wid = cid * NS + sid
    pltpu.sync_copy(src_hbm.at[wid], srcb_v)
    pltpu.sync_copy(dst_hbm.at[wid], dstb_v)
    pltpu.sync_copy(zeros_hbm, zero_v)

    def gstart(idx, rows, ch, off):
        def mk(j, c):
            idx[pl.ds(j * 16, 16)] = srcb_v[ch, pl.ds(j * 16, 16)] + off
            return c
        lax.fori_loop(0, CH // 16, mk, 0)
        pltpu.async_copy(t_hbm.at[idx], rows, sem)

    def gwait(rows):
        pltpu.make_async_copy(t_hbm.at[pl.ds(0, CH)], rows, sem).wait()

    for p in range(P):
        for h in range(2):
            off = 2 * p + h
            pltpu.sync_copy(zero_v, acc_sh.at[pl.ds(sid * RPT, RPT)])
            plsc.subcore_barrier()

            gstart(idx_a, rows_a, 0, off)

            def pair(i, c, off=off):
                ch = 2 * i
                gwait(rows_a)
                gstart(idx_b, rows_b, ch + 1, off)
                pltpu.sync_copy(rows_a, acc_sh.at[dstb_v.at[ch]], add=True)
                gwait(rows_b)
                gstart(idx_a, rows_a, ch + 2, off)
                pltpu.sync_copy(rows_b, acc_sh.at[dstb_v.at[ch + 1]],
                                add=True)
                return c
            lax.fori_loop(0, (CPW - 1) // 2, pair, 0)

            gwait(rows_a)
            pltpu.sync_copy(rows_a, acc_sh.at[dstb_v.at[CPW - 1]], add=True)
            plsc.subcore_barrier()

            pltpu.sync_copy(acc_sh.at[pl.ds(sid * RPT, RPT)],
                            out_hbm.at[cid, p, h, pl.ds(sid * RPT, RPT)])
            plsc.subcore_barrier()


def _tc_project(xt2, w_all, dinv12):
    RB = 2400

    def body(x_ref, w_ref, d_ref, o_ref):
        t = jnp.dot(x_ref[...], w_ref[...], preferred_element_type=jnp.float32)
        o_ref[...] = t * d_ref[...]

    return pl.pallas_call(
        body,
        grid=(xt2.shape[0] // RB,),
        in_specs=[
            pl.BlockSpec((RB, F_IN), lambda i: (i, 0)),
            pl.BlockSpec((F_IN, FW), lambda i: (0, 0)),
            pl.BlockSpec((RB, 1), lambda i: (i, 0)),
        ],
        out_specs=pl.BlockSpec((RB, FW), lambda i: (i, 0)),
        out_shape=jax.ShapeDtypeStruct((xt2.shape[0], FW), jnp.float32),
    )(xt2, w_all, dinv12)


def _tc_gru(partials, dinv2, att2, ball, U_z, c_z, U_r, c_r, U_h, c_h,
            W_lin, b_lin):
    NB = 1000

    def body(pp_ref, d_ref, att_ref, ball_ref, uz_ref, cz_ref, ur_ref,
             cr_ref, uh_ref, ch_ref, wl_ref, bl_ref, o_ref):
        probs = jax.nn.softmax(att_ref[...], axis=-1)
        dinv = d_ref[...]
        H = jnp.zeros((NB, F_OUT), jnp.float32)
        Hacc = jnp.zeros((NB, F_OUT), jnp.float32)
        for p in range(P):
            raw = jnp.concatenate(
                [pp_ref[0, p, 0] + pp_ref[1, p, 0],
                 pp_ref[0, p, 1] + pp_ref[1, p, 1]], axis=1)
            C = raw * dinv + ball_ref[...]
            Cz = C[:, 0:F_OUT]
            Cr = C[:, F_OUT:2 * F_OUT]
            Chh = C[:, 2 * F_OUT:3 * F_OUT]
            Z = jax.nn.sigmoid(
                jnp.dot(jnp.concatenate([Cz, H], axis=1), uz_ref[...],
                        preferred_element_type=jnp.float32) + cz_ref[...])
            R = jax.nn.sigmoid(
                jnp.dot(jnp.concatenate([Cr, H], axis=1), ur_ref[...],
                        preferred_element_type=jnp.float32) + cr_ref[...])
            Ht = jnp.tanh(
                jnp.dot(jnp.concatenate([Chh, H * R], axis=1), uh_ref[...],
                        preferred_element_type=jnp.float32) + ch_ref[...])
            H = Z * H + (1.0 - Z) * Ht
            Hacc = Hacc + probs[0:1, p:p + 1] * H
        o_ref[...] = (jnp.dot(jnp.maximum(Hacc, 0.0), wl_ref[...],
                              preferred_element_type=jnp.float32)
                      + bl_ref[...])

    return pl.pallas_call(
        body,
        grid=(N // NB,),
        in_specs=[
            pl.BlockSpec((NC, P, 2, NB, HW), lambda i: (0, 0, 0, i, 0)),
            pl.BlockSpec((NB, 1), lambda i: (i, 0)),
            pl.BlockSpec((1, P), lambda i: (0, 0)),
            pl.BlockSpec((1, FW), lambda i: (0, 0)),
            pl.BlockSpec((2 * F_OUT, F_OUT), lambda i: (0, 0)),
            pl.BlockSpec((1, F_OUT), lambda i: (0, 0)),
            pl.BlockSpec((2 * F_OUT, F_OUT), lambda i: (0, 0)),
            pl.BlockSpec((1, F_OUT), lambda i: (0, 0)),
            pl.BlockSpec((2 * F_OUT, F_OUT), lambda i: (0, 0)),
            pl.BlockSpec((1, F_OUT), lambda i: (0, 0)),
            pl.BlockSpec((F_OUT, P), lambda i: (0, 0)),
            pl.BlockSpec((1, P), lambda i: (0, 0)),
        ],
        out_specs=pl.BlockSpec((NB, P), lambda i: (i, 0)),
        out_shape=jax.ShapeDtypeStruct((N, P), jnp.float32),
    )(partials, dinv2, att2, ball, U_z, c_z, U_r, c_r, U_h, c_h,
      W_lin, b_lin)


def kernel(x, edge_index, attention, W_z, b_z, W_r, b_r, W_h, b_h,
           U_z, c_z, U_r, c_r, U_h, c_h, W_lin, b_lin):
    src = edge_index[0].astype(jnp.int32)
    dst = edge_index[1].astype(jnp.int32)
    loop = jnp.arange(N, dtype=jnp.int32)
    pad = EP - E - N
    src_f = jnp.concatenate([src, loop, jnp.zeros((pad,), jnp.int32)])
    dst_pad = N + jnp.arange(pad, dtype=jnp.int32) % (NPAD - N)
    dst_f = jnp.concatenate([dst, loop, dst_pad])
    src24 = jnp.swapaxes((src_f * (2 * P)).reshape(CPW, W, CH), 0, 1)
    dstb = jnp.swapaxes(dst_f.reshape(CPW, W, CH), 0, 1)

    degp = _deg_kernel(dstb)
    deg = degp[0, :N, 0] + degp[1, :N, 0]
    dinv = lax.rsqrt(deg)
    dinv12 = jnp.repeat(dinv, P)[:, None]

    w_all = jnp.concatenate([W_z, W_r, W_h], axis=1)
    xt2 = jnp.swapaxes(x, 1, 2).reshape(N * P, F_IN)
    t_tab = _tc_project(xt2, w_all, dinv12)
    t_half = t_tab.reshape(N * P * 2, HW)

    zeros_rpt = jnp.zeros((RPT, HW), jnp.float32)
    partials = _agg_kernel(t_half, src24, dstb, zeros_rpt)

    ball = jnp.concatenate([b_z, b_r, b_h])[None, :]
    out = _tc_gru(partials, dinv[:, None], attention[None, :], ball,
                  U_z, c_z[None, :], U_r, c_r[None, :], U_h, c_h[None, :],
                  W_lin, b_lin[None, :])
    return out


# speedup vs baseline: 3.1151x; 1.3027x over previous
import functools

import jax
import jax.numpy as jnp
from jax import lax
from jax.experimental import pallas as pl
from jax.experimental.pallas import tpu as pltpu
from jax.experimental.pallas import tpu_sc as plsc

N = 10000
E = 320000
F_IN = 128
F_OUT = 32
P = 12
FW = 3 * F_OUT

NC, NS = 2, 16
W = NC * NS
CH = 128
CPW = 81
EPW = CPW * CH
EP = W * EPW
NPAD = 10240
RPT = NPAD // NS

_mesh = plsc.VectorSubcoreMesh(core_axis_name="c", subcore_axis_name="s")


@functools.partial(
    pl.kernel,
    mesh=_mesh,
    compiler_params=pltpu.CompilerParams(use_tc_tiling_on_sc=False),
    out_type=jax.ShapeDtypeStruct((NC, NPAD, 16), jnp.float32),
    scratch_types=[
        pltpu.VMEM((CPW, CH), jnp.int32),
        pltpu.VMEM((CH, 16), jnp.float32),
        pltpu.VMEM((RPT, 16), jnp.float32),
        pltpu.VMEM_SHARED((NPAD, 16), jnp.float32),
    ],
)
def _deg_kernel(dst_hbm, out_hbm, dstb_v, ones_v, zero_v, acc_sh):
    cid = lax.axis_index("c")
    sid = lax.axis_index("s")
    wid = cid * NS + sid
    pltpu.sync_copy(dst_hbm.at[wid], dstb_v)

    def fill_ones(i, c):
        ones_v[i, :] = jnp.ones((16,), jnp.float32)
        return c
    lax.fori_loop(0, CH, fill_ones, 0)

    def fill_zero(i, c):
        zero_v[i, :] = jnp.zeros((16,), jnp.float32)
        return c
    lax.fori_loop(0, RPT, fill_zero, 0)

    pltpu.sync_copy(zero_v, acc_sh.at[pl.ds(sid * RPT, RPT)])
    plsc.subcore_barrier()

    def chunk(ch, c):
        pltpu.sync_copy(ones_v, acc_sh.at[dstb_v.at[ch]], add=True)
        return c
    lax.fori_loop(0, CPW, chunk, 0)
    plsc.subcore_barrier()

    pltpu.sync_copy(acc_sh.at[pl.ds(sid * RPT, RPT)],
                    out_hbm.at[cid, pl.ds(sid * RPT, RPT)])


HW = FW // 2


@functools.partial(
    pl.kernel,
    mesh=_mesh,
    compiler_params=pltpu.CompilerParams(use_tc_tiling_on_sc=False),
    out_type=jax.ShapeDtypeStruct((NC, P, 2, NPAD, HW), jnp.float32),
    scratch_types=[
        pltpu.VMEM((CPW, CH), jnp.int32),
        pltpu.VMEM((CPW, CH), jnp.int32),
        pltpu.VMEM((CH,), jnp.int32),
        pltpu.VMEM((CH,), jnp.int32),
        pltpu.VMEM((CH,), jnp.int32),
        pltpu.VMEM((CH, HW), jnp.float32),
        pltpu.VMEM((CH, HW), jnp.float32),
        pltpu.VMEM((CH, HW), jnp.float32),
        pltpu.VMEM((RPT, HW), jnp.float32),
        pltpu.VMEM_SHARED((NPAD, HW), jnp.float32),
        pltpu.SemaphoreType.DMA,
    ],
)
def _agg_kernel(t_hbm, src_hbm, dst_hbm, zeros_hbm, out_hbm,
                srcb_v, dstb_v, idx_a, idx_b, idx_c, rows_a, rows_b, rows_c,
                zero_v, acc_sh, sem):
    cid = lax.axis_index("c")
    sid = lax.axis_index("s")
    wid = cid * NS + sid
    pltpu.sync_copy(src_hbm.at[wid], srcb_v)
    pltpu.sync_copy(dst_hbm.at[wid], dstb_v)
    pltpu.sync_copy(zeros_hbm, zero_v)

    def gstart(idx, rows, ch, off):
        def mk(j, c):
            idx[pl.ds(j * 16, 16)] = srcb_v[ch, pl.ds(j * 16, 16)] + off
            return c
        lax.fori_loop(0, CH // 16, mk, 0)
        pltpu.async_copy(t_hbm.at[idx], rows, sem)

    def gwait(rows):
        pltpu.make_async_copy(t_hbm.at[pl.ds(0, CH)], rows, sem).wait()

    for p in range(P):
        for h in range(2):
            off = 2 * p + h
            pltpu.sync_copy(zero_v, acc_sh.at[pl.ds(sid * RPT, RPT)])
            plsc.subcore_barrier()

            gstart(idx_a, rows_a, 0, off)
            gstart(idx_b, rows_b, 1, off)
            gstart(idx_c, rows_c, 2, off)

            slots = ((idx_a, rows_a), (idx_b, rows_b), (idx_c, rows_c))

            def triple(i, c, off=off):
                ch = 3 * i
                for k, (idx, rows) in enumerate(slots):
                    gwait(rows)
                    pltpu.sync_copy(rows, acc_sh.at[dstb_v.at[ch + k]],
                                    add=True)
                    gstart(idx, rows, ch + k + 3, off)
                return c
            lax.fori_loop(0, CPW // 3 - 1, triple, 0)

            for k, (idx, rows) in enumerate(slots):
                gwait(rows)
                pltpu.sync_copy(rows, acc_sh.at[dstb_v.at[CPW - 3 + k]],
                                add=True)
            plsc.subcore_barrier()

            pltpu.sync_copy(acc_sh.at[pl.ds(sid * RPT, RPT)],
                            out_hbm.at[cid, p, h, pl.ds(sid * RPT, RPT)])
            plsc.subcore_barrier()


def _tc_project(xt2, w_all, dinv12):
    RB = 2400

    def body(x_ref, w_ref, d_ref, o_ref):
        t = jnp.dot(x_ref[...], w_ref[...], preferred_element_type=jnp.float32)
        o_ref[...] = t * d_ref[...]

    return pl.pallas_call(
        body,
        grid=(xt2.shape[0] // RB,),
        in_specs=[
            pl.BlockSpec((RB, F_IN), lambda i: (i, 0)),
            pl.BlockSpec((F_IN, FW), lambda i: (0, 0)),
            pl.BlockSpec((RB, 1), lambda i: (i, 0)),
        ],
        out_specs=pl.BlockSpec((RB, FW), lambda i: (i, 0)),
        out_shape=jax.ShapeDtypeStruct((xt2.shape[0], FW), jnp.float32),
    )(xt2, w_all, dinv12)


def _tc_gru(partials, dinv2, att2, ball, U_z, c_z, U_r, c_r, U_h, c_h,
            W_lin, b_lin):
    NB = 1000

    def body(pp_ref, d_ref, att_ref, ball_ref, uz_ref, cz_ref, ur_ref,
             cr_ref, uh_ref, ch_ref, wl_ref, bl_ref, o_ref):
        probs = jax.nn.softmax(att_ref[...], axis=-1)
        dinv = d_ref[...]
        H = jnp.zeros((NB, F_OUT), jnp.float32)
        Hacc = jnp.zeros((NB, F_OUT), jnp.float32)
        for p in range(P):
            raw = jnp.concatenate(
                [pp_ref[0, p, 0] + pp_ref[1, p, 0],
                 pp_ref[0, p, 1] + pp_ref[1, p, 1]], axis=1)
            C = raw * dinv + ball_ref[...]
            Cz = C[:, 0:F_OUT]
            Cr = C[:, F_OUT:2 * F_OUT]
            Chh = C[:, 2 * F_OUT:3 * F_OUT]
            Z = jax.nn.sigmoid(
                jnp.dot(jnp.concatenate([Cz, H], axis=1), uz_ref[...],
                        preferred_element_type=jnp.float32) + cz_ref[...])
            R = jax.nn.sigmoid(
                jnp.dot(jnp.concatenate([Cr, H], axis=1), ur_ref[...],
                        preferred_element_type=jnp.float32) + cr_ref[...])
            Ht = jnp.tanh(
                jnp.dot(jnp.concatenate([Chh, H * R], axis=1), uh_ref[...],
                        preferred_element_type=jnp.float32) + ch_ref[...])
            H = Z * H + (1.0 - Z) * Ht
            Hacc = Hacc + probs[0:1, p:p + 1] * H
        o_ref[...] = (jnp.dot(jnp.maximum(Hacc, 0.0), wl_ref[...],
                              preferred_element_type=jnp.float32)
                      + bl_ref[...])

    return pl.pallas_call(
        body,
        grid=(N // NB,),
        in_specs=[
            pl.BlockSpec((NC, P, 2, NB, HW), lambda i: (0, 0, 0, i, 0)),
            pl.BlockSpec((NB, 1), lambda i: (i, 0)),
            pl.BlockSpec((1, P), lambda i: (0, 0)),
            pl.BlockSpec((1, FW), lambda i: (0, 0)),
            pl.BlockSpec((2 * F_OUT, F_OUT), lambda i: (0, 0)),
            pl.BlockSpec((1, F_OUT), lambda i: (0, 0)),
            pl.BlockSpec((2 * F_OUT, F_OUT), lambda i: (0, 0)),
            pl.BlockSpec((1, F_OUT), lambda i: (0, 0)),
            pl.BlockSpec((2 * F_OUT, F_OUT), lambda i: (0, 0)),
            pl.BlockSpec((1, F_OUT), lambda i: (0, 0)),
            pl.BlockSpec((F_OUT, P), lambda i: (0, 0)),
            pl.BlockSpec((1, P), lambda i: (0, 0)),
        ],
        out_specs=pl.BlockSpec((NB, P), lambda i: (i, 0)),
        out_shape=jax.ShapeDtypeStruct((N, P), jnp.float32),
    )(partials, dinv2, att2, ball, U_z, c_z, U_r, c_r, U_h, c_h,
      W_lin, b_lin)


def kernel(x, edge_index, attention, W_z, b_z, W_r, b_r, W_h, b_h,
           U_z, c_z, U_r, c_r, U_h, c_h, W_lin, b_lin):
    src = edge_index[0].astype(jnp.int32)
    dst = edge_index[1].astype(jnp.int32)
    loop = jnp.arange(N, dtype=jnp.int32)
    pad = EP - E - N
    src_f = jnp.concatenate([src, loop, jnp.zeros((pad,), jnp.int32)])
    dst_pad = N + jnp.arange(pad, dtype=jnp.int32) % (NPAD - N)
    dst_f = jnp.concatenate([dst, loop, dst_pad])
    src24 = jnp.swapaxes((src_f * (2 * P)).reshape(CPW, W, CH), 0, 1)
    dstb = jnp.swapaxes(dst_f.reshape(CPW, W, CH), 0, 1)

    degp = _deg_kernel(dstb)
    deg = degp[0, :N, 0] + degp[1, :N, 0]
    dinv = lax.rsqrt(deg)
    dinv12 = jnp.repeat(dinv, P)[:, None]

    w_all = jnp.concatenate([W_z, W_r, W_h], axis=1)
    xt2 = jnp.swapaxes(x, 1, 2).reshape(N * P, F_IN)
    t_tab = _tc_project(xt2, w_all, dinv12)
    t_half = t_tab.reshape(N * P * 2, HW)

    zeros_rpt = jnp.zeros((RPT, HW), jnp.float32)
    partials = _agg_kernel(t_half, src24, dstb, zeros_rpt)

    ball = jnp.concatenate([b_z, b_r, b_h])[None, :]
    out = _tc_gru(partials, dinv[:, None], attention[None, :], ball,
                  U_z, c_z[None, :], U_r, c_r[None, :], U_h, c_h[None, :],
                  W_lin, b_lin[None, :])
    return out
